# baseline (speedup 1.0000x reference)
import functools

import jax
import jax.numpy as jnp
from jax import lax
from jax.experimental import pallas as pl
from jax.experimental.pallas import tpu as pltpu
from jax.experimental.pallas import tpu_sc as plsc

NBINS = 64
COUT = 32
K = 3
H = 384
HO = H - K + 1
MAP_E = HO * HO
NMAPS = 16 * COUT
NW = 32
MAPS_PER_W = NMAPS // NW
W_E = MAPS_PER_W * MAP_E
CHUNK = 16384
NFULL = W_E // CHUNK
TAIL = W_E - NFULL * CHUNK
HIST_W = MAPS_PER_W * NBINS


def _conv_idx_kernel(x_ref, w_ref, b_ref, out_ref):
    c = pl.program_id(1)
    acc = jnp.zeros((HO, HO), dtype=jnp.float32)
    for di in range(K):
        for dj in range(K):
            acc = acc + w_ref[c, di * K + dj] * x_ref[0, di:di + HO, dj:dj + HO]
    y = jnp.maximum(acc + b_ref[c], 0.0)
    lo = jnp.min(y)
    hi = jnp.max(y)
    same = hi == lo
    lo = jnp.where(same, lo - 1.0, lo)
    hi = jnp.where(same, hi + 1.0, hi)
    scale = NBINS / (hi - lo)
    idx = jnp.floor((y - lo) * scale).astype(jnp.int32)
    idx = jnp.clip(idx, 0, NBINS - 1)
    out_ref[0] = idx + lax.rem(c, MAPS_PER_W) * NBINS


def _sc_hist(idx_hbm, out_hbm, buf0, buf1, hist, merged, sem0, sem1):
    wid = lax.axis_index("s") * 2 + lax.axis_index("c")
    base = wid * W_E
    lanebase = lax.iota(jnp.int32, 16) * HIST_W
    ones = jnp.ones((16,), jnp.float32)
    zeros = jnp.zeros((16,), jnp.float32)

    def zero_body(i, _):
        hist[pl.ds(i * 16, 16)] = zeros
        return 0

    lax.fori_loop(0, 16 * HIST_W // 16, zero_body, 0)

    def process(buf, nwords):
        def body(i, _):
            v = buf[pl.ds(i * 16, 16)]
            plsc.addupdate_scatter(hist, [lanebase + v], ones)
            return 0

        lax.fori_loop(0, nwords // 16, body, 0, unroll=8)

    pltpu.async_copy(idx_hbm.at[pl.ds(base, CHUNK)], buf0, sem0)

    def pair_body(p, _):
        off = base + 2 * p * CHUNK
        c1 = pltpu.async_copy(idx_hbm.at[pl.ds(off + CHUNK, CHUNK)], buf1, sem1)
        pltpu.make_async_copy(idx_hbm.at[pl.ds(off, CHUNK)], buf0, sem0).wait()
        process(buf0, CHUNK)

        @pl.when(p < NFULL // 2 - 1)
        def _():
            pltpu.async_copy(
                idx_hbm.at[pl.ds(off + 2 * CHUNK, CHUNK)], buf0, sem0)

        c1.wait()
        process(buf1, CHUNK)
        return 0

    lax.fori_loop(0, NFULL // 2, pair_body, 0)

    pltpu.sync_copy(
        idx_hbm.at[pl.ds(base + NFULL * CHUNK, TAIL)], buf0.at[pl.ds(0, TAIL)])
    process(buf0, TAIL)

    def merge_body(j, _):
        acc = hist[pl.ds(j * 16, 16)]
        for l in range(1, 16):
            acc = acc + hist[pl.ds(l * HIST_W + j * 16, 16)]
        merged[pl.ds(j * 16, 16)] = acc
        return 0

    lax.fori_loop(0, HIST_W // 16, merge_body, 0)
    pltpu.sync_copy(merged, out_hbm.at[pl.ds(wid * HIST_W, HIST_W)])


def _head_kernel(h_ref, w_ref, b_ref, o_ref):
    o_ref[...] = (
        jnp.dot(h_ref[...], w_ref[...], preferred_element_type=jnp.float32)
        + b_ref[...].reshape(1, -1))


def kernel(x, conv_w, conv_b, head_w, head_b):
    B = x.shape[0]
    FC = head_w.shape[0]
    xs = x.reshape(B, H, H)
    wf = conv_w.reshape(COUT, K * K)

    idx_maps = pl.pallas_call(
        _conv_idx_kernel,
        grid=(B, COUT),
        in_specs=[
            pl.BlockSpec((1, H, H), lambda b, c: (b, 0, 0)),
            pl.BlockSpec(memory_space=pltpu.SMEM),
            pl.BlockSpec(memory_space=pltpu.SMEM),
        ],
        out_specs=pl.BlockSpec((1, HO, HO), lambda b, c: (b * COUT + c, 0, 0)),
        out_shape=jax.ShapeDtypeStruct((NMAPS, HO, HO), jnp.int32),
    )(xs, wf, conv_b)

    sc_hist = functools.partial(
        pl.kernel,
        mesh=plsc.VectorSubcoreMesh(core_axis_name="c", subcore_axis_name="s"),
        compiler_params=pltpu.CompilerParams(needs_layout_passes=False),
        out_type=jax.ShapeDtypeStruct((NMAPS * NBINS,), jnp.float32),
        scratch_types=[
            pltpu.VMEM((CHUNK,), jnp.int32),
            pltpu.VMEM((CHUNK,), jnp.int32),
            pltpu.VMEM((16 * HIST_W,), jnp.float32),
            pltpu.VMEM((HIST_W,), jnp.float32),
            pltpu.SemaphoreType.DMA,
            pltpu.SemaphoreType.DMA,
        ],
    )(_sc_hist)

    counts = sc_hist(idx_maps.reshape(-1))
    h = counts.reshape(B, COUT * NBINS)

    out = pl.pallas_call(
        _head_kernel,
        in_specs=[
            pl.BlockSpec((B, COUT * NBINS), lambda: (0, 0)),
            pl.BlockSpec((COUT * NBINS, FC), lambda: (0, 0)),
            pl.BlockSpec((FC,), lambda: (0,)),
        ],
        out_specs=pl.BlockSpec((B, FC), lambda: (0, 0)),
        out_shape=jax.ShapeDtypeStruct((B, FC), jnp.float32),
    )(h, head_w.T, head_b)
    return out


# speedup vs baseline: 1.1462x; 1.1462x over previous
import functools

import jax
import jax.numpy as jnp
from jax import lax
from jax.experimental import pallas as pl
from jax.experimental.pallas import tpu as pltpu
from jax.experimental.pallas import tpu_sc as plsc

NBINS = 64
COUT = 32
K = 3
H = 384
HO = H - K + 1
MAP_E = HO * HO
NMAPS = 16 * COUT
NW = 32
MAPS_PER_W = NMAPS // NW
W_E = MAPS_PER_W * MAP_E
CHUNK = 16384
NFULL = W_E // CHUNK
TAIL = W_E - NFULL * CHUNK
HIST_W = MAPS_PER_W * NBINS


def _conv_idx_kernel(x_ref, w_ref, b_ref, out_ref):
    c = pl.program_id(1)
    acc = jnp.zeros((HO, HO), dtype=jnp.float32)
    for di in range(K):
        for dj in range(K):
            acc = acc + w_ref[c, di * K + dj] * x_ref[0, di:di + HO, dj:dj + HO]
    y = jnp.maximum(acc + b_ref[c], 0.0)
    lo = jnp.min(y)
    hi = jnp.max(y)
    same = hi == lo
    lo = jnp.where(same, lo - 1.0, lo)
    hi = jnp.where(same, hi + 1.0, hi)
    scale = NBINS / (hi - lo)
    idx = jnp.floor((y - lo) * scale).astype(jnp.int32)
    idx = jnp.clip(idx, 0, NBINS - 1)
    out_ref[0] = idx + lax.rem(c, MAPS_PER_W) * NBINS


def _sc_hist(idx_hbm, out_hbm, buf0, buf1, hist, merged, sem0, sem1):
    wid = lax.axis_index("s") * 2 + lax.axis_index("c")
    base = wid * W_E
    lanebase = lax.iota(jnp.int32, 16) * HIST_W
    ones = jnp.ones((16,), jnp.float32)
    zeros = jnp.zeros((16,), jnp.float32)

    def zero_body(i, _):
        hist[pl.ds(i * 16, 16)] = zeros
        return 0

    lax.fori_loop(0, 16 * HIST_W // 16, zero_body, 0)

    def process(buf, nwords):
        @plsc.parallel_loop(0, nwords // 16, unroll=8)
        def _(i):
            v = buf[pl.ds(i * 16, 16)]
            plsc.addupdate_scatter(hist, [lanebase + v], ones)

    pltpu.async_copy(idx_hbm.at[pl.ds(base, CHUNK)], buf0, sem0)

    def pair_body(p, _):
        off = base + 2 * p * CHUNK
        c1 = pltpu.async_copy(idx_hbm.at[pl.ds(off + CHUNK, CHUNK)], buf1, sem1)
        pltpu.make_async_copy(idx_hbm.at[pl.ds(off, CHUNK)], buf0, sem0).wait()
        process(buf0, CHUNK)

        @pl.when(p < NFULL // 2 - 1)
        def _():
            pltpu.async_copy(
                idx_hbm.at[pl.ds(off + 2 * CHUNK, CHUNK)], buf0, sem0)

        c1.wait()
        process(buf1, CHUNK)
        return 0

    lax.fori_loop(0, NFULL // 2, pair_body, 0)

    pltpu.sync_copy(
        idx_hbm.at[pl.ds(base + NFULL * CHUNK, TAIL)], buf0.at[pl.ds(0, TAIL)])
    process(buf0, TAIL)

    def merge_body(j, _):
        acc = hist[pl.ds(j * 16, 16)]
        for l in range(1, 16):
            acc = acc + hist[pl.ds(l * HIST_W + j * 16, 16)]
        merged[pl.ds(j * 16, 16)] = acc
        return 0

    lax.fori_loop(0, HIST_W // 16, merge_body, 0)
    pltpu.sync_copy(merged, out_hbm.at[pl.ds(wid * HIST_W, HIST_W)])


def _head_kernel(h_ref, w_ref, b_ref, o_ref):
    o_ref[...] = (
        jnp.dot(h_ref[...], w_ref[...], preferred_element_type=jnp.float32)
        + b_ref[...].reshape(1, -1))


def kernel(x, conv_w, conv_b, head_w, head_b):
    B = x.shape[0]
    FC = head_w.shape[0]
    xs = x.reshape(B, H, H)
    wf = conv_w.reshape(COUT, K * K)

    idx_maps = pl.pallas_call(
        _conv_idx_kernel,
        grid=(B, COUT),
        in_specs=[
            pl.BlockSpec((1, H, H), lambda b, c: (b, 0, 0)),
            pl.BlockSpec(memory_space=pltpu.SMEM),
            pl.BlockSpec(memory_space=pltpu.SMEM),
        ],
        out_specs=pl.BlockSpec((1, HO, HO), lambda b, c: (b * COUT + c, 0, 0)),
        out_shape=jax.ShapeDtypeStruct((NMAPS, HO, HO), jnp.int32),
    )(xs, wf, conv_b)

    sc_hist = functools.partial(
        pl.kernel,
        mesh=plsc.VectorSubcoreMesh(core_axis_name="c", subcore_axis_name="s"),
        compiler_params=pltpu.CompilerParams(needs_layout_passes=False),
        out_type=jax.ShapeDtypeStruct((NMAPS * NBINS,), jnp.float32),
        scratch_types=[
            pltpu.VMEM((CHUNK,), jnp.int32),
            pltpu.VMEM((CHUNK,), jnp.int32),
            pltpu.VMEM((16 * HIST_W,), jnp.float32),
            pltpu.VMEM((HIST_W,), jnp.float32),
            pltpu.SemaphoreType.DMA,
            pltpu.SemaphoreType.DMA,
        ],
    )(_sc_hist)

    counts = sc_hist(idx_maps.reshape(-1))
    h = counts.reshape(B, COUT * NBINS)

    out = pl.pallas_call(
        _head_kernel,
        in_specs=[
            pl.BlockSpec((B, COUT * NBINS), lambda: (0, 0)),
            pl.BlockSpec((COUT * NBINS, FC), lambda: (0, 0)),
            pl.BlockSpec((FC,), lambda: (0,)),
        ],
        out_specs=pl.BlockSpec((B, FC), lambda: (0, 0)),
        out_shape=jax.ShapeDtypeStruct((B, FC), jnp.float32),
    )(h, head_w.T, head_b)
    return out


# speedup vs baseline: 3.6908x; 3.2199x over previous
import functools

import jax
import jax.numpy as jnp
from jax import lax
from jax.experimental import pallas as pl
from jax.experimental.pallas import tpu as pltpu
from jax.experimental.pallas import tpu_sc as plsc

NBINS = 64
COUT = 32
K = 3
H = 384
HO = H - K + 1
NMAPS = 16 * COUT
TILES_PER_MAP = (H // 8) * (H // 128)
NTILES = NMAPS * TILES_PER_MAP
NW = 32
MAPS_PER_W = NMAPS // NW
W_TILES = MAPS_PER_W * TILES_PER_MAP
CHUNK_T = 32
NCHUNK = W_TILES // CHUNK_T
TRASH = MAPS_PER_W * NBINS
LANE_STRIDE = TRASH + 16


def _conv_idx_kernel(x_ref, w_ref, b_ref, out_ref):
    c = pl.program_id(1)
    acc = jnp.zeros((HO, HO), dtype=jnp.float32)
    for di in range(K):
        for dj in range(K):
            acc = acc + w_ref[c, di * K + dj] * x_ref[0, di:di + HO, dj:dj + HO]
    y = jnp.maximum(acc + b_ref[c], 0.0)
    lo = jnp.min(y)
    hi = jnp.max(y)
    same = hi == lo
    lo = jnp.where(same, lo - 1.0, lo)
    hi = jnp.where(same, hi + 1.0, hi)
    scale = NBINS / (hi - lo)
    idx = jnp.floor((y - lo) * scale).astype(jnp.int32)
    idx = jnp.clip(idx, 0, NBINS - 1)
    idx = idx + lax.rem(c, MAPS_PER_W) * NBINS
    idx384 = jnp.pad(idx, ((0, H - HO), (0, H - HO)), constant_values=TRASH)
    out_ref[...] = (
        idx384.reshape(H // 8, 8, H // 128, 128)
        .transpose(0, 2, 1, 3)
        .reshape(TILES_PER_MAP, 8, 128))


def _sc_hist(idx_hbm, out_hbm, buf0, buf1, hist, merged, sem0, sem1):
    wid = lax.axis_index("s") * 2 + lax.axis_index("c")
    base = wid * W_TILES
    lanebase = lax.iota(jnp.int32, 16) * LANE_STRIDE
    ones = jnp.ones((16,), jnp.float32)
    zeros = jnp.zeros((16,), jnp.float32)

    def zero_body(i, _):
        hist[pl.ds(i * 16, 16)] = zeros
        return 0

    lax.fori_loop(0, 16 * LANE_STRIDE // 16, zero_body, 0)

    def process(buf):
        @plsc.parallel_loop(0, CHUNK_T, unroll=2)
        def _(t):
            for s in range(8):
                for l in range(8):
                    v = buf[t, s, pl.ds(l * 16, 16)]
                    plsc.addupdate_scatter(hist, [lanebase + v], ones)

    pltpu.async_copy(idx_hbm.at[pl.ds(base, CHUNK_T)], buf0, sem0)

    def pair_body(p, _):
        off = base + 2 * p * CHUNK_T
        c1 = pltpu.async_copy(
            idx_hbm.at[pl.ds(off + CHUNK_T, CHUNK_T)], buf1, sem1)
        pltpu.make_async_copy(
            idx_hbm.at[pl.ds(off, CHUNK_T)], buf0, sem0).wait()
        process(buf0)

        @pl.when(p < NCHUNK // 2 - 1)
        def _():
            pltpu.async_copy(
                idx_hbm.at[pl.ds(off + 2 * CHUNK_T, CHUNK_T)], buf0, sem0)

        c1.wait()
        process(buf1)
        return 0

    lax.fori_loop(0, NCHUNK // 2, pair_body, 0)

    def merge_body(j, _):
        acc = hist[pl.ds(j * 16, 16)]
        for l in range(1, 16):
            acc = acc + hist[pl.ds(l * LANE_STRIDE + j * 16, 16)]
        merged[pl.ds(j * 16, 16)] = acc
        return 0

    lax.fori_loop(0, TRASH // 16, merge_body, 0)
    pltpu.sync_copy(merged, out_hbm.at[pl.ds(wid * TRASH, TRASH)])


def _head_kernel(h_ref, w_ref, b_ref, o_ref):
    o_ref[...] = (
        jnp.dot(h_ref[...], w_ref[...], preferred_element_type=jnp.float32)
        + b_ref[...].reshape(1, -1))


def kernel(x, conv_w, conv_b, head_w, head_b):
    B = x.shape[0]
    FC = head_w.shape[0]
    xs = x.reshape(B, H, H)
    wf = conv_w.reshape(COUT, K * K)

    idx_tiles = pl.pallas_call(
        _conv_idx_kernel,
        grid=(B, COUT),
        in_specs=[
            pl.BlockSpec((1, H, H), lambda b, c: (b, 0, 0)),
            pl.BlockSpec(memory_space=pltpu.SMEM),
            pl.BlockSpec(memory_space=pltpu.SMEM),
        ],
        out_specs=pl.BlockSpec(
            (TILES_PER_MAP, 8, 128),
            lambda b, c: (b * COUT + c, 0, 0)),
        out_shape=jax.ShapeDtypeStruct((NTILES, 8, 128), jnp.int32),
    )(xs, wf, conv_b)

    sc_hist = functools.partial(
        pl.kernel,
        mesh=plsc.VectorSubcoreMesh(core_axis_name="c", subcore_axis_name="s"),
        compiler_params=pltpu.CompilerParams(needs_layout_passes=False),
        out_type=jax.ShapeDtypeStruct((NMAPS * NBINS,), jnp.float32),
        scratch_types=[
            pltpu.VMEM((CHUNK_T, 8, 128), jnp.int32),
            pltpu.VMEM((CHUNK_T, 8, 128), jnp.int32),
            pltpu.VMEM((16 * LANE_STRIDE,), jnp.float32),
            pltpu.VMEM((TRASH,), jnp.float32),
            pltpu.SemaphoreType.DMA,
            pltpu.SemaphoreType.DMA,
        ],
    )(_sc_hist)

    counts = sc_hist(idx_tiles)
    h = counts.reshape(B, COUT * NBINS)

    out = pl.pallas_call(
        _head_kernel,
        in_specs=[
            pl.BlockSpec((B, COUT * NBINS), lambda: (0, 0)),
            pl.BlockSpec((COUT * NBINS, FC), lambda: (0, 0)),
            pl.BlockSpec((FC,), lambda: (0,)),
        ],
        out_specs=pl.BlockSpec((B, FC), lambda: (0, 0)),
        out_shape=jax.ShapeDtypeStruct((B, FC), jnp.float32),
    )(h, head_w.T, head_b)
    return out


# speedup vs baseline: 4.4101x; 1.1949x over previous
import functools

import jax
import jax.numpy as jnp
from jax import lax
from jax.experimental import pallas as pl
from jax.experimental.pallas import tpu as pltpu
from jax.experimental.pallas import tpu_sc as plsc

NBINS = 64
COUT = 32
K = 3
H = 384
HO = H - K + 1
NMAPS = 16 * COUT
TILES_PER_MAP = (H // 8) * (H // 128)
NTILES = NMAPS * TILES_PER_MAP
NW = 32
MAPS_PER_W = NMAPS // NW
W_TILES = MAPS_PER_W * TILES_PER_MAP
CHUNK_T = 32
NCHUNK = W_TILES // CHUNK_T
TRASH = MAPS_PER_W * NBINS
LANE_STRIDE = TRASH + 17


def _conv_idx_kernel(x_ref, w_ref, b_ref, out_ref):
    c = pl.program_id(1)
    acc = jnp.zeros((HO, HO), dtype=jnp.float32)
    for di in range(K):
        for dj in range(K):
            acc = acc + w_ref[c, di * K + dj] * x_ref[0, di:di + HO, dj:dj + HO]
    y = jnp.maximum(acc + b_ref[c], 0.0)
    lo = jnp.min(y)
    hi = jnp.max(y)
    same = hi == lo
    lo = jnp.where(same, lo - 1.0, lo)
    hi = jnp.where(same, hi + 1.0, hi)
    scale = NBINS / (hi - lo)
    idx = jnp.floor((y - lo) * scale).astype(jnp.int32)
    idx = jnp.clip(idx, 0, NBINS - 1)
    idx = idx + lax.rem(c, MAPS_PER_W) * NBINS
    idx384 = jnp.pad(idx, ((0, H - HO), (0, H - HO)), constant_values=TRASH)
    out_ref[...] = (
        idx384.reshape(H // 8, 8, H // 128, 128)
        .transpose(0, 2, 1, 3)
        .reshape(TILES_PER_MAP, 8, 128))


def _sc_hist(idx_hbm, out_hbm, buf0, buf1, hist, merged, sem0, sem1):
    wid = lax.axis_index("s") * 2 + lax.axis_index("c")
    base = wid * W_TILES
    lanebase = lax.iota(jnp.int32, 16) * LANE_STRIDE
    ones = jnp.ones((16,), jnp.float32)
    zeros = jnp.zeros((16,), jnp.float32)

    def zero_body(i, _):
        hist[pl.ds(i * 16, 16)] = zeros
        return 0

    lax.fori_loop(0, 16 * LANE_STRIDE // 16, zero_body, 0)

    def process(buf):
        @plsc.parallel_loop(0, CHUNK_T, unroll=2)
        def _(t):
            for s in range(8):
                for l in range(8):
                    v = buf[t, s, pl.ds(l * 16, 16)]
                    plsc.addupdate_scatter(hist, [lanebase + v], ones)

    pltpu.async_copy(idx_hbm.at[pl.ds(base, CHUNK_T)], buf0, sem0)

    def pair_body(p, _):
        off = base + 2 * p * CHUNK_T
        c1 = pltpu.async_copy(
            idx_hbm.at[pl.ds(off + CHUNK_T, CHUNK_T)], buf1, sem1)
        pltpu.make_async_copy(
            idx_hbm.at[pl.ds(off, CHUNK_T)], buf0, sem0).wait()
        process(buf0)

        @pl.when(p < NCHUNK // 2 - 1)
        def _():
            pltpu.async_copy(
                idx_hbm.at[pl.ds(off + 2 * CHUNK_T, CHUNK_T)], buf0, sem0)

        c1.wait()
        process(buf1)
        return 0

    lax.fori_loop(0, NCHUNK // 2, pair_body, 0)

    def merge_body(j, _):
        acc = hist[pl.ds(j * 16, 16)]
        for l in range(1, 16):
            acc = acc + hist[pl.ds(l * LANE_STRIDE + j * 16, 16)]
        merged[pl.ds(j * 16, 16)] = acc
        return 0

    lax.fori_loop(0, TRASH // 16, merge_body, 0)
    pltpu.sync_copy(merged, out_hbm.at[pl.ds(wid * TRASH, TRASH)])


def _head_kernel(h_ref, w_ref, b_ref, o_ref):
    o_ref[...] = (
        jnp.dot(h_ref[...], w_ref[...], preferred_element_type=jnp.float32)
        + b_ref[...].reshape(1, -1))


def kernel(x, conv_w, conv_b, head_w, head_b):
    B = x.shape[0]
    FC = head_w.shape[0]
    xs = x.reshape(B, H, H)
    wf = conv_w.reshape(COUT, K * K)

    idx_tiles = pl.pallas_call(
        _conv_idx_kernel,
        grid=(B, COUT),
        in_specs=[
            pl.BlockSpec((1, H, H), lambda b, c: (b, 0, 0)),
            pl.BlockSpec(memory_space=pltpu.SMEM),
            pl.BlockSpec(memory_space=pltpu.SMEM),
        ],
        out_specs=pl.BlockSpec(
            (TILES_PER_MAP, 8, 128),
            lambda b, c: (b * COUT + c, 0, 0)),
        out_shape=jax.ShapeDtypeStruct((NTILES, 8, 128), jnp.int32),
    )(xs, wf, conv_b)

    sc_hist = functools.partial(
        pl.kernel,
        mesh=plsc.VectorSubcoreMesh(core_axis_name="c", subcore_axis_name="s"),
        compiler_params=pltpu.CompilerParams(needs_layout_passes=False),
        out_type=jax.ShapeDtypeStruct((NMAPS * NBINS,), jnp.float32),
        scratch_types=[
            pltpu.VMEM((CHUNK_T, 8, 128), jnp.int32),
            pltpu.VMEM((CHUNK_T, 8, 128), jnp.int32),
            pltpu.VMEM((16 * LANE_STRIDE,), jnp.float32),
            pltpu.VMEM((TRASH,), jnp.float32),
            pltpu.SemaphoreType.DMA,
            pltpu.SemaphoreType.DMA,
        ],
    )(_sc_hist)

    counts = sc_hist(idx_tiles)
    h = counts.reshape(B, COUT * NBINS)

    out = pl.pallas_call(
        _head_kernel,
        in_specs=[
            pl.BlockSpec((B, COUT * NBINS), lambda: (0, 0)),
            pl.BlockSpec((COUT * NBINS, FC), lambda: (0, 0)),
            pl.BlockSpec((FC,), lambda: (0,)),
        ],
        out_specs=pl.BlockSpec((B, FC), lambda: (0, 0)),
        out_shape=jax.ShapeDtypeStruct((B, FC), jnp.float32),
    )(h, head_w.T, head_b)
    return out


# speedup vs baseline: 4.8031x; 1.0891x over previous
import functools

import jax
import jax.numpy as jnp
from jax import lax
from jax.experimental import pallas as pl
from jax.experimental.pallas import tpu as pltpu
from jax.experimental.pallas import tpu_sc as plsc

NBINS = 64
COUT = 32
K = 3
H = 384
HO = H - K + 1
NMAPS = 16 * COUT
TILES_PER_MAP = (H // 8) * (H // 128)
NTILES = NMAPS * TILES_PER_MAP
NW = 32
MAPS_PER_W = NMAPS // NW
W_TILES = MAPS_PER_W * TILES_PER_MAP
CHUNK_T = 32
NCHUNK = W_TILES // CHUNK_T
TRASH = MAPS_PER_W * NBINS
LANE_STRIDE = TRASH + 17


def _conv_idx_kernel(x_ref, w_ref, b_ref, out_ref, scr_ref):
    c = pl.program_id(1)

    @pl.when(c == 0)
    def _():
        for di in range(K):
            for dj in range(K):
                scr_ref[di * K + dj, :, 0:H - dj] = x_ref[0, di:di + HO, dj:H]

    acc = jnp.zeros((HO, H), dtype=jnp.float32)
    for k in range(K * K):
        acc = acc + w_ref[c, k] * scr_ref[k]
    y = jnp.maximum(acc + b_ref[c], 0.0)
    colmask = lax.broadcasted_iota(jnp.int32, (HO, H), 1) < HO
    lo = jnp.min(jnp.where(colmask, y, jnp.inf))
    hi = jnp.max(jnp.where(colmask, y, -jnp.inf))
    same = hi == lo
    lo = jnp.where(same, lo - 1.0, lo)
    hi = jnp.where(same, hi + 1.0, hi)
    scale = NBINS / (hi - lo)
    idx = jnp.floor((y - lo) * scale).astype(jnp.int32)
    idx = jnp.clip(idx, 0, NBINS - 1)
    idx = idx + lax.rem(c, MAPS_PER_W) * NBINS
    idx = jnp.where(colmask, idx, TRASH)
    idx384 = jnp.pad(idx, ((0, H - HO), (0, 0)), constant_values=TRASH)
    out_ref[...] = (
        idx384.reshape(H // 8, 8, H // 128, 128)
        .transpose(0, 2, 1, 3)
        .reshape(TILES_PER_MAP, 8, 128))


def _sc_hist(idx_hbm, out_hbm, buf0, buf1, hist, merged, sem0, sem1):
    wid = lax.axis_index("s") * 2 + lax.axis_index("c")
    base = wid * W_TILES
    lanebase = lax.iota(jnp.int32, 16) * LANE_STRIDE
    ones = jnp.ones((16,), jnp.float32)
    zeros = jnp.zeros((16,), jnp.float32)

    def zero_body(i, _):
        hist[pl.ds(i * 16, 16)] = zeros
        return 0

    lax.fori_loop(0, 16 * LANE_STRIDE // 16, zero_body, 0)

    def process(buf):
        @plsc.parallel_loop(0, CHUNK_T, unroll=4)
        def _(t):
            for s in range(8):
                for l in range(8):
                    v = buf[t, s, pl.ds(l * 16, 16)]
                    plsc.addupdate_scatter(hist, [lanebase + v], ones)

    pltpu.async_copy(idx_hbm.at[pl.ds(base, CHUNK_T)], buf0, sem0)

    def pair_body(p, _):
        off = base + 2 * p * CHUNK_T
        c1 = pltpu.async_copy(
            idx_hbm.at[pl.ds(off + CHUNK_T, CHUNK_T)], buf1, sem1)
        pltpu.make_async_copy(
            idx_hbm.at[pl.ds(off, CHUNK_T)], buf0, sem0).wait()
        process(buf0)

        @pl.when(p < NCHUNK // 2 - 1)
        def _():
            pltpu.async_copy(
                idx_hbm.at[pl.ds(off + 2 * CHUNK_T, CHUNK_T)], buf0, sem0)

        c1.wait()
        process(buf1)
        return 0

    lax.fori_loop(0, NCHUNK // 2, pair_body, 0)

    def merge_body(j, _):
        acc = hist[pl.ds(j * 16, 16)]
        for l in range(1, 16):
            acc = acc + hist[pl.ds(l * LANE_STRIDE + j * 16, 16)]
        merged[pl.ds(j * 16, 16)] = acc
        return 0

    lax.fori_loop(0, TRASH // 16, merge_body, 0)
    pltpu.sync_copy(merged, out_hbm.at[pl.ds(wid * TRASH, TRASH)])


def _head_kernel(h_ref, w_ref, b_ref, o_ref):
    o_ref[...] = (
        jnp.dot(h_ref[...], w_ref[...], preferred_element_type=jnp.float32)
        + b_ref[...].reshape(1, -1))


def kernel(x, conv_w, conv_b, head_w, head_b):
    B = x.shape[0]
    FC = head_w.shape[0]
    xs = x.reshape(B, H, H)
    wf = conv_w.reshape(COUT, K * K)

    idx_tiles = pl.pallas_call(
        _conv_idx_kernel,
        grid=(B, COUT),
        in_specs=[
            pl.BlockSpec((1, H, H), lambda b, c: (b, 0, 0)),
            pl.BlockSpec(memory_space=pltpu.SMEM),
            pl.BlockSpec(memory_space=pltpu.SMEM),
        ],
        out_specs=pl.BlockSpec(
            (TILES_PER_MAP, 8, 128),
            lambda b, c: (b * COUT + c, 0, 0)),
        out_shape=jax.ShapeDtypeStruct((NTILES, 8, 128), jnp.int32),
        scratch_shapes=[pltpu.VMEM((K * K, HO, H), jnp.float32)],
    )(xs, wf, conv_b)

    sc_hist = functools.partial(
        pl.kernel,
        mesh=plsc.VectorSubcoreMesh(core_axis_name="c", subcore_axis_name="s"),
        compiler_params=pltpu.CompilerParams(needs_layout_passes=False),
        out_type=jax.ShapeDtypeStruct((NMAPS * NBINS,), jnp.float32),
        scratch_types=[
            pltpu.VMEM((CHUNK_T, 8, 128), jnp.int32),
            pltpu.VMEM((CHUNK_T, 8, 128), jnp.int32),
            pltpu.VMEM((16 * LANE_STRIDE,), jnp.float32),
            pltpu.VMEM((TRASH,), jnp.float32),
            pltpu.SemaphoreType.DMA,
            pltpu.SemaphoreType.DMA,
        ],
    )(_sc_hist)

    counts = sc_hist(idx_tiles)
    h = counts.reshape(B, COUT * NBINS)

    out = pl.pallas_call(
        _head_kernel,
        in_specs=[
            pl.BlockSpec((B, COUT * NBINS), lambda: (0, 0)),
            pl.BlockSpec((COUT * NBINS, FC), lambda: (0, 0)),
            pl.BlockSpec((FC,), lambda: (0,)),
        ],
        out_specs=pl.BlockSpec((B, FC), lambda: (0, 0)),
        out_shape=jax.ShapeDtypeStruct((B, FC), jnp.float32),
    )(h, head_w.T, head_b)
    return out
